# Initial kernel scaffold; baseline (speedup 1.0000x reference)
import jax
import jax.numpy as jnp
from jax.experimental import pallas as pl


def kernel(x, pe):
    raise NotImplementedError("write your pallas kernel here")


# speedup vs baseline: 3.4015x; 3.4015x over previous
import jax
import jax.numpy as jnp
from jax.experimental import pallas as pl

_TS = 1024


def _add_pe_kernel(x_ref, pe_ref, o_ref):
    o_ref[...] = x_ref[...] + pe_ref[...]


def kernel(x, pe):
    B, S, D = x.shape
    ts = _TS if S % _TS == 0 else S
    grid = (S // ts, B)
    return pl.pallas_call(
        _add_pe_kernel,
        grid=grid,
        in_specs=[
            pl.BlockSpec((1, ts, D), lambda s, b: (b, s, 0)),
            pl.BlockSpec((ts, D), lambda s, b: (s, 0)),
        ],
        out_specs=pl.BlockSpec((1, ts, D), lambda s, b: (b, s, 0)),
        out_shape=jax.ShapeDtypeStruct((B, S, D), x.dtype),
    )(x, pe[:S])


# speedup vs baseline: 3.6251x; 1.0657x over previous
import jax
import jax.numpy as jnp
from jax.experimental import pallas as pl

_TS = 2048


def _add_pe_kernel(x_ref, pe_ref, o_ref):
    o_ref[...] = x_ref[...] + pe_ref[...]


def kernel(x, pe):
    B, S, D = x.shape
    ts = _TS if S % _TS == 0 else S
    grid = (S // ts, B)
    return pl.pallas_call(
        _add_pe_kernel,
        grid=grid,
        in_specs=[
            pl.BlockSpec((1, ts, D), lambda s, b: (b, s, 0)),
            pl.BlockSpec((ts, D), lambda s, b: (s, 0)),
        ],
        out_specs=pl.BlockSpec((1, ts, D), lambda s, b: (b, s, 0)),
        out_shape=jax.ShapeDtypeStruct((B, S, D), x.dtype),
    )(x, pe[:S])


# speedup vs baseline: 3.6448x; 1.0054x over previous
import jax
import jax.numpy as jnp
from jax.experimental import pallas as pl

_TS = 1024


def _add_pe_kernel(x_ref, pe_ref, o_ref):
    o_ref[...] = x_ref[...] + pe_ref[...][None, :, :]


def kernel(x, pe):
    B, S, D = x.shape
    ts = _TS if S % _TS == 0 else S
    grid = (S // ts,)
    return pl.pallas_call(
        _add_pe_kernel,
        grid=grid,
        in_specs=[
            pl.BlockSpec((B, ts, D), lambda s: (0, s, 0)),
            pl.BlockSpec((ts, D), lambda s: (s, 0)),
        ],
        out_specs=pl.BlockSpec((B, ts, D), lambda s: (0, s, 0)),
        out_shape=jax.ShapeDtypeStruct((B, S, D), x.dtype),
    )(x, pe[:S])
